# Initial kernel scaffold; baseline (speedup 1.0000x reference)
import jax
import jax.numpy as jnp
from jax.experimental import pallas as pl


def kernel(x, memory, Wq, bq, Wk, bk, Wv, bv, Wg, bg):
    raise NotImplementedError("write your pallas kernel here")


# speedup vs baseline: 10.1181x; 10.1181x over previous
import jax
import jax.numpy as jnp
from jax.experimental import pallas as pl
from jax.experimental.pallas import tpu as pltpu

_B, _S, _D, _M, _K = 16, 20, 64, 16384, 8
_HI = 128
_LO = _M // _HI
_C = _S * _K
_SCALE = float(_D) ** 0.5

_DEF = jax.lax.Precision.DEFAULT
_HIP = jax.lax.Precision.HIGHEST


def _dot(a, b, dims, prec=_DEF):
    return jax.lax.dot_general(a, b, dimension_numbers=(dims, ((), ())),
                               preferred_element_type=jnp.float32,
                               precision=prec)


def _bdot(a, b, cdims, bdims, prec=_DEF):
    return jax.lax.dot_general(a, b, dimension_numbers=(cdims, bdims),
                               preferred_element_type=jnp.float32,
                               precision=prec)


def _b16(v):
    return v.astype(jnp.bfloat16)


def _body(x_ref, m0_ref, m1_ref, wq_ref, bq_ref, wk_ref, bk_ref,
          wv_ref, bv_ref, wg_ref, bg_ref, out_ref, kb_ref, vb_ref):
    wq = wq_ref[...]
    wk = wk_ref[...]
    wv = wv_ref[...]
    wg = wg_ref[...]
    bq = bq_ref[...]
    bk = bk_ref[...]
    bv = bv_ref[...]
    bg = bg_ref[...]

    kb_ref[...] = _b16(_dot(m0_ref[...], wk, ((1,), (0,))) + bk)
    vb_ref[...] = _b16(_dot(m0_ref[...], wv, ((1,), (0,))) + bv)

    def step(t, carry):
        sidx, sval = carry
        iota_m = jax.lax.broadcasted_iota(jnp.int32, (_B, _M), 1)
        iota_c = jax.lax.broadcasted_iota(jnp.int32, (_B, _C, _HI), 2)
        slot_iota = jax.lax.broadcasted_iota(jnp.int32, (_B, _C), 1)
        slot_iota3 = jax.lax.broadcasted_iota(jnp.int32, (_B, _C, _D), 1)
        xt = x_ref[:, pl.ds(t, 1), :].reshape(_B, _D)
        q = _dot(xt, wq, ((1,), (0,))) + bq
        qs = q * (1.0 / _SCALE)
        lb = _dot(qs, kb_ref[...], ((1,), (1,)))

        valid = sidx < _M
        vmaskf = valid.astype(jnp.float32)
        kc16 = _b16(_dot(sval, wk, ((2,), (0,))) + bk)
        qsb = _b16(qs).astype(jnp.float32)
        lcorr = jnp.sum(kc16.astype(jnp.float32) * qsb[:, None, :], axis=2)

        hi = jax.lax.shift_right_logical(sidx, 7)
        lo = jnp.bitwise_and(sidx, _LO - 1)
        onehot_hi = (hi[:, :, None] == iota_c).astype(jnp.float32)
        onehot_lo = (lo[:, :, None] == iota_c).astype(jnp.float32)
        ah = onehot_hi * vmaskf[:, :, None]
        mask2 = _bdot(ah, onehot_lo, ((1,), (1,)), ((0,), (0,))).reshape(_B, _M)
        scat2 = _bdot(ah * lcorr[:, :, None], onehot_lo,
                      ((1,), (1,)), ((0,), (0,)), _HIP).reshape(_B, _M)
        lfull = lb * (1.0 - mask2) + scat2

        rm = jnp.max(lfull, axis=1)
        e2 = jnp.exp(lfull - rm[:, None])
        a_s = jnp.where(valid,
                        jnp.exp(jnp.where(valid, lcorr - rm[:, None], -30.0)),
                        0.0)
        denom = jnp.sum(e2 * (1.0 - mask2), axis=1) + jnp.sum(a_s, axis=1)
        att0 = e2 * (1.0 - mask2) / denom[:, None]
        aat = (a_s / denom[:, None])
        vc16 = _b16(_dot(sval, wv, ((2,), (0,))) + bv)
        mem_out = (_dot(att0, vb_ref[...], ((1,), (0,)))
                   + jnp.sum(_b16(aat).astype(jnp.float32)[:, :, None]
                             * vc16.astype(jnp.float32), axis=1))

        g = jax.nn.sigmoid(_dot(xt, wg[:_D], ((1,), (0,)))
                           + _dot(mem_out, wg[_D:], ((1,), (0,))) + bg)

        lwork = lfull
        idxs = []
        for _ in range(_K):
            mx = jnp.max(lwork, axis=1)
            cand = jnp.where(lwork == mx[:, None], iota_m, _M)
            idxj = jnp.min(cand, axis=1)
            lwork = jnp.where(iota_m == idxj[:, None], -jnp.inf, lwork)
            idxs.append(idxj)

        new_sidx = sidx
        new_sval = sval
        for j, idxj in enumerate(idxs):
            match = valid & (sidx == idxj[:, None])
            matchf = match.astype(jnp.float32)
            foundf = jnp.sum(matchf, axis=1)
            old_store = jnp.sum(matchf[:, :, None] * sval, axis=1)
            ohm = (iota_m == idxj[:, None]).astype(jnp.bfloat16)
            old_mem = (_dot(ohm, m0_ref[...], ((1,), (0,)))
                       + _dot(ohm, m1_ref[...], ((1,), (0,))))
            old = old_store + (1.0 - foundf)[:, None] * old_mem
            new = (1.0 - g) * old + g * xt
            new_sidx = jnp.where(new_sidx == idxj[:, None], _M, new_sidx)
            wmask = slot_iota == (_K * t + j)
            new_sidx = jnp.where(wmask, idxj[:, None], new_sidx)
            new_sval = jnp.where(slot_iota3 == (_K * t + j),
                                 new[:, None, :], new_sval)

        out_ref[:, pl.ds(t, 1), :] = mem_out[:, None, :]
        return new_sidx, new_sval

    sidx0 = jnp.full((_B, _C), _M, jnp.int32)
    sval0 = jnp.zeros((_B, _C, _D), jnp.float32)
    jax.lax.fori_loop(0, _S, step, (sidx0, sval0))


def kernel(x, memory, Wq, bq, Wk, bk, Wv, bv, Wg, bg):
    m0 = memory.astype(jnp.bfloat16)
    m1 = (memory - m0.astype(jnp.float32)).astype(jnp.bfloat16)
    return pl.pallas_call(
        _body,
        out_shape=jax.ShapeDtypeStruct((_B, _S, _D), jnp.float32),
        scratch_shapes=[pltpu.VMEM((_M, _D), jnp.bfloat16),
                        pltpu.VMEM((_M, _D), jnp.bfloat16)],
        compiler_params=pltpu.CompilerParams(
            vmem_limit_bytes=100 * 1024 * 1024),
    )(x, m0, m1, Wq, bq.reshape(1, _D), Wk, bk.reshape(1, _D),
      Wv, bv.reshape(1, _D), Wg, bg.reshape(1, _D))
